# Initial kernel scaffold; baseline (speedup 1.0000x reference)
import jax
import jax.numpy as jnp
from jax.experimental import pallas as pl


def kernel(image, x):
    raise NotImplementedError("write your pallas kernel here")


# speedup vs baseline: 6.7139x; 6.7139x over previous
import jax
import jax.numpy as jnp
from jax import lax
from jax.experimental import pallas as pl
from jax.experimental.pallas import tpu as pltpu
from jax.experimental.pallas import tpu_sc as plsc

N = 262144
C = 64
GRID = 64
TABLE_ROWS = GRID * GRID

_NC = 2
_NS = 16
_NW = _NC * _NS
_L = 16

_B = 128
_PER_W = N // _NW
_CHUNKS = _PER_W // _B


def _body(table_hbm, x_hbm, out_hbm, x_v, idx_v, w_v, rows_v, sem):
    wid = lax.axis_index("s") * _NC + lax.axis_index("c")
    base = wid * _PER_W

    lanes = lax.iota(jnp.int32, _L)
    zeros = jnp.zeros((_L,), jnp.int32)
    ones = jnp.ones((_L,), jnp.int32)
    cap = jnp.full((_L,), GRID - 1, jnp.int32)

    def chunk(ci, carry):
        off = base + ci * _B
        pltpu.sync_copy(x_hbm.at[pl.ds(2 * off, 2 * _B)], x_v)
        for j in range(_B // _L):
            pair_ids = (lanes + (j * _L)) * 2
            x0 = plsc.load_gather(x_v, [pair_ids])
            x1 = plsc.load_gather(x_v, [pair_ids + 1])
            low0 = x0.astype(jnp.int32)
            low1 = x1.astype(jnp.int32)
            w0 = (low0 + 1).astype(jnp.float32) - x0
            w1 = (low1 + 1).astype(jnp.float32) - x1
            idx = jnp.minimum(low0, cap) * GRID + jnp.minimum(low1, cap)
            idx_v[pl.ds(j * _L, _L)] = idx
            w_v[pl.ds(j * _L, _L)] = w0 * w1
        pltpu.async_copy(table_hbm.at[idx_v], rows_v, sem).wait()

        def scale(b, carry2):
            wb = plsc.load_gather(w_v, [zeros + b])
            for cc in range(C // _L):
                seg = rows_v[b, pl.ds(cc * _L, _L)]
                rows_v[b, pl.ds(cc * _L, _L)] = seg * wb
            return carry2

        lax.fori_loop(0, _B, scale, 0)
        pltpu.sync_copy(rows_v, out_hbm.at[pl.ds(off, _B)])
        return carry

    lax.fori_loop(0, _CHUNKS, chunk, 0)


_kern = pl.kernel(
    _body,
    out_type=jax.ShapeDtypeStruct((N, C), jnp.float32),
    mesh=plsc.VectorSubcoreMesh(core_axis_name="c", subcore_axis_name="s"),
    scratch_types=[
        pltpu.VMEM((2 * _B,), jnp.float32),
        pltpu.VMEM((_B,), jnp.int32),
        pltpu.VMEM((_B,), jnp.float32),
        pltpu.VMEM((_B, C), jnp.float32),
        pltpu.SemaphoreType.DMA,
    ],
    compiler_params=pltpu.CompilerParams(
        needs_layout_passes=False, use_tc_tiling_on_sc=False
    ),
)


def kernel(image, x):
    return _kern(image.reshape(TABLE_ROWS, C), x.reshape(2 * N))


# speedup vs baseline: 7.1021x; 1.0578x over previous
import jax
import jax.numpy as jnp
from jax import lax
from jax.experimental import pallas as pl
from jax.experimental.pallas import tpu as pltpu
from jax.experimental.pallas import tpu_sc as plsc

N = 262144
C = 64
GRID = 64
TABLE_ROWS = GRID * GRID

_NC = 2
_NS = 16
_NW = _NC * _NS
_L = 16

_B = 128
_PER_W = N // _NW
_CHUNKS = _PER_W // _B


def _body(table_hbm, x_hbm, out_hbm, x_v, idx_v, w_v, rows0, rows1,
          gsem0, gsem1, osem0, osem1):
    wid = lax.axis_index("s") * _NC + lax.axis_index("c")
    base = wid * _PER_W

    pltpu.sync_copy(x_hbm.at[pl.ds(2 * base, 2 * _PER_W)], x_v)

    lanes = lax.iota(jnp.int32, _L)
    cap = jnp.full((_L,), GRID - 1, jnp.int32)

    def comp(i, carry):
        for j in range(_B // _L):
            q0 = i * _B + j * _L
            pair = 2 * q0 + 2 * lanes
            x0 = plsc.load_gather(x_v, [pair])
            x1 = plsc.load_gather(x_v, [pair + 1])
            low0 = x0.astype(jnp.int32)
            low1 = x1.astype(jnp.int32)
            w0 = (low0 + 1).astype(jnp.float32) - x0
            w1 = (low1 + 1).astype(jnp.float32) - x1
            idx_v[pl.ds(q0, _L)] = jnp.minimum(low0, cap) * GRID + jnp.minimum(low1, cap)
            w_v[pl.ds(q0, _L)] = w0 * w1
        return carry

    lax.fori_loop(0, _CHUNKS, comp, 0)

    rows = (rows0, rows1)
    gsem = (gsem0, gsem1)
    osem = (osem0, osem1)

    def fire_gather(ci, p):
        pltpu.async_copy(
            table_hbm.at[idx_v.at[pl.ds(ci * _B, _B)]], rows[p], gsem[p])

    fire_gather(0, 0)

    def pair_body(cp, carry):
        for par in (0, 1):
            ci = 2 * cp + par
            q = 1 - par

            @pl.when(ci >= 1)
            def _wait_out():
                pltpu.make_async_copy(
                    rows[q], out_hbm.at[pl.ds(0, _B)], osem[q]).wait()

            @pl.when(ci + 1 < _CHUNKS)
            def _next_gather():
                fire_gather(ci + 1, q)

            pltpu.make_async_copy(
                table_hbm.at[pl.ds(0, _B)], rows[par], gsem[par]).wait()

            woff = ci * _B
            zero = jnp.zeros((_L,), jnp.int32)
            for b in range(_B):
                wb = plsc.load_gather(w_v, [zero + (woff + b)])
                for cc in range(C // _L):
                    seg = rows[par][b, pl.ds(cc * _L, _L)]
                    rows[par][b, pl.ds(cc * _L, _L)] = seg * wb

            pltpu.async_copy(
                rows[par], out_hbm.at[pl.ds(base + ci * _B, _B)], osem[par])
        return carry

    lax.fori_loop(0, _CHUNKS // 2, pair_body, 0)

    pltpu.make_async_copy(rows1, out_hbm.at[pl.ds(0, _B)], osem1).wait()


_kern = pl.kernel(
    _body,
    out_type=jax.ShapeDtypeStruct((N, C), jnp.float32),
    mesh=plsc.VectorSubcoreMesh(core_axis_name="c", subcore_axis_name="s"),
    scratch_types=[
        pltpu.VMEM((2 * _PER_W,), jnp.float32),
        pltpu.VMEM((_PER_W,), jnp.int32),
        pltpu.VMEM((_PER_W,), jnp.float32),
        pltpu.VMEM((_B, C), jnp.float32),
        pltpu.VMEM((_B, C), jnp.float32),
        pltpu.SemaphoreType.DMA,
        pltpu.SemaphoreType.DMA,
        pltpu.SemaphoreType.DMA,
        pltpu.SemaphoreType.DMA,
    ],
    compiler_params=pltpu.CompilerParams(
        needs_layout_passes=False, use_tc_tiling_on_sc=False
    ),
)


def kernel(image, x):
    return _kern(image.reshape(TABLE_ROWS, C), x.reshape(2 * N))
